# baseline (speedup 1.0000x reference)
import functools

import jax
import jax.numpy as jnp
from jax.experimental import pallas as pl
from jax.experimental.pallas import tpu as pltpu


def _conv_stats_scatter_kernel(xp_ref, w_ref, p_ref, y_ref, stats_ref,
                               *, rt, W, Cin, Cout):
    i0 = pl.program_id(1) * rt
    W2 = 2 * W
    ssum = jnp.zeros((1, Cout), jnp.float32)
    qsum = jnp.zeros((1, Cout), jnp.float32)
    accs = []
    for a in range(2):
        for b in range(2):
            acc = jnp.zeros((rt * W, Cout), jnp.float32)
            for r in range(2):
                for s in range(2):
                    xs = xp_ref[s + b, 0, pl.ds(i0 + r + a, rt), :, :]
                    xf = xs.reshape(rt * W, Cin)
                    acc = acc + jnp.dot(xf, w_ref[a * 8 + b * 4 + r * 2 + s],
                                        preferred_element_type=jnp.float32)
            ssum = ssum + jnp.sum(acc, axis=0, keepdims=True)
            qsum = qsum + jnp.sum(acc * acc, axis=0, keepdims=True)
            accs.append(acc.astype(jnp.bfloat16))
    ys = []
    for g in range(rt // 2):
        lhs = jnp.concatenate([ab[2 * W * g:2 * W * (g + 1)] for ab in accs],
                              axis=0)
        ys.append(jax.lax.dot_general(lhs, p_ref[...],
                                      (((0,), (0,)), ((), ())),
                                      preferred_element_type=jnp.float32))
    stats_ref[0] = jnp.concatenate([ssum, qsum], axis=0)
    y_ref[0] = jnp.concatenate(ys, axis=1).astype(jnp.bfloat16)


def _affine_relu_kernel(y_ref, sc_ref, sh_ref, out_ref):
    sc = jnp.transpose(sc_ref[...])
    sh = jnp.transpose(sh_ref[...])
    y = y_ref[0].astype(jnp.float32) * sc + sh
    out_ref[0] = jnp.maximum(y, 0.0).astype(jnp.bfloat16)


def kernel(x, skip, conv_w, gamma, beta):
    N, Cin, H, W = x.shape
    Cout = conv_w.shape[0]
    Cskip = skip.shape[1]
    H2, W2 = 2 * H, 2 * W
    rt = 32 if H % 32 == 0 else 8
    T = H // rt
    lanes = 2 * W2
    eps = 0.8
    pad_size = 2

    xp = jnp.pad(jnp.transpose(x, (0, 2, 3, 1)),
                 ((0, 0), (1, 1), (1, 1), (0, 0)))
    xp3 = jnp.stack([xp[:, :, 0:W], xp[:, :, 1:W + 1], xp[:, :, 2:W + 2]],
                    axis=0).astype(jnp.bfloat16)
    sel = jnp.array([[[1.0, 0.0, 0.0], [0.0, 1.0, 1.0]],
                     [[1.0, 1.0, 0.0], [0.0, 0.0, 1.0]]], dtype=jnp.float32)
    wf = jnp.einsum("arh,bsw,oihw->abrsio", sel, sel,
                    conv_w.astype(jnp.float32))
    wf = wf.reshape(16, Cin, Cout).astype(jnp.bfloat16)

    rr = jnp.arange(4 * 2 * W)
    ph, i_loc, qj = rr // (2 * W), (rr % (2 * W)) // W, rr % W
    col = i_loc * lanes + (ph // 2) * W2 + 2 * qj + (ph % 2)
    pmat = jax.nn.one_hot(col, 2 * lanes, dtype=jnp.bfloat16)

    grid = (N, T)
    cparams = pltpu.CompilerParams(
        dimension_semantics=("parallel", "parallel"),
        vmem_limit_bytes=48 * 1024 * 1024,
    )

    y_pre, stats = pl.pallas_call(
        functools.partial(_conv_stats_scatter_kernel,
                          rt=rt, W=W, Cin=Cin, Cout=Cout),
        out_shape=(jax.ShapeDtypeStruct((N, Cout, T * rt * lanes),
                                        jnp.bfloat16),
                   jax.ShapeDtypeStruct((N * T, 2, Cout), jnp.float32)),
        grid=grid,
        in_specs=[pl.BlockSpec((3, 1, H + 2, W, Cin),
                               lambda n, t: (0, n, 0, 0, 0)),
                  pl.BlockSpec((16, Cin, Cout), lambda n, t: (0, 0, 0)),
                  pl.BlockSpec((4 * 2 * W, 2 * lanes), lambda n, t: (0, 0))],
        out_specs=(pl.BlockSpec((1, Cout, rt * lanes), lambda n, t: (n, 0, t)),
                   pl.BlockSpec((1, 2, Cout), lambda n, t: (n * T + t, 0, 0))),
        compiler_params=cparams,
    )(xp3, wf, pmat)

    m = float(N * H2 * W2)
    mean = jnp.sum(stats[:, 0, :], axis=0) / m
    var = jnp.maximum(jnp.sum(stats[:, 1, :], axis=0) / m - mean * mean, 0.0)
    scale = gamma.astype(jnp.float32) * jax.lax.rsqrt(var + eps)
    shift = beta.astype(jnp.float32) - mean * scale

    P = T * rt * lanes
    yb = pl.pallas_call(
        _affine_relu_kernel,
        out_shape=jax.ShapeDtypeStruct((N, Cout, P), jnp.bfloat16),
        grid=(N,),
        in_specs=[pl.BlockSpec((1, Cout, P), lambda n: (n, 0, 0)),
                  pl.BlockSpec((1, Cout), lambda n: (0, 0)),
                  pl.BlockSpec((1, Cout), lambda n: (0, 0))],
        out_specs=pl.BlockSpec((1, Cout, P), lambda n: (n, 0, 0)),
        compiler_params=pltpu.CompilerParams(
            dimension_semantics=("parallel",),
            vmem_limit_bytes=48 * 1024 * 1024,
        ),
    )(y_pre, scale.reshape(1, Cout), shift.reshape(1, Cout))
    y4 = yb.reshape(N, Cout, H2, W2).astype(jnp.float32)
    Hs, Ws = skip.shape[2], skip.shape[3]
    skip_p = jnp.concatenate(
        [jnp.zeros((N, Cskip, Hs, pad_size), skip.dtype), skip], axis=3)
    skip_p = jnp.concatenate(
        [skip_p, jnp.zeros((N, Cskip, pad_size, W2), skip.dtype)], axis=2)
    return jnp.concatenate([y4, skip_p], axis=1)


# speedup vs baseline: 1.0643x; 1.0643x over previous
import functools

import jax
import jax.numpy as jnp
from jax.experimental import pallas as pl
from jax.experimental.pallas import tpu as pltpu


def _conv_stats_scatter_kernel(xp_ref, w_ref, p_ref, y_ref, stats_ref,
                               *, rt, W, Cin, Cout):
    i0 = pl.program_id(1) * rt
    W2 = 2 * W
    G = y_ref.shape[0]
    for img in range(G):
        ssum = jnp.zeros((1, Cout), jnp.float32)
        qsum = jnp.zeros((1, Cout), jnp.float32)
        accs = []
        for a in range(2):
            for b in range(2):
                acc = jnp.zeros((rt * W, Cout), jnp.float32)
                for r in range(2):
                    for s in range(2):
                        xs = xp_ref[s + b, img, pl.ds(i0 + r + a, rt), :, :]
                        xf = xs.reshape(rt * W, Cin)
                        acc = acc + jnp.dot(
                            xf, w_ref[a * 8 + b * 4 + r * 2 + s],
                            preferred_element_type=jnp.float32)
                ssum = ssum + jnp.sum(acc, axis=0, keepdims=True)
                qsum = qsum + jnp.sum(acc * acc, axis=0, keepdims=True)
                accs.append(acc.astype(jnp.bfloat16))
        ys = []
        for g in range(rt // 2):
            lhs = jnp.concatenate(
                [ab[2 * W * g:2 * W * (g + 1)] for ab in accs], axis=0)
            ys.append(jax.lax.dot_general(lhs, p_ref[...],
                                          (((0,), (0,)), ((), ())),
                                          preferred_element_type=jnp.float32))
        stats_ref[img] = jnp.concatenate([ssum, qsum], axis=0)
        y_ref[img] = jnp.concatenate(ys, axis=1).astype(jnp.bfloat16)


def kernel(x, skip, conv_w, gamma, beta):
    N, Cin, H, W = x.shape
    Cout = conv_w.shape[0]
    Cskip = skip.shape[1]
    H2, W2 = 2 * H, 2 * W
    rt = 32 if H % 32 == 0 else 8
    T = H // rt
    lanes = 2 * W2
    eps = 0.8
    pad_size = 2

    xp = jnp.pad(jnp.transpose(x, (0, 2, 3, 1)),
                 ((0, 0), (1, 1), (1, 1), (0, 0)))
    xp3 = jnp.stack([xp[:, :, 0:W], xp[:, :, 1:W + 1], xp[:, :, 2:W + 2]],
                    axis=0).astype(jnp.bfloat16)
    sel = jnp.array([[[1.0, 0.0, 0.0], [0.0, 1.0, 1.0]],
                     [[1.0, 1.0, 0.0], [0.0, 0.0, 1.0]]], dtype=jnp.float32)
    wf = jnp.einsum("arh,bsw,oihw->abrsio", sel, sel,
                    conv_w.astype(jnp.float32))
    wf = wf.reshape(16, Cin, Cout).astype(jnp.bfloat16)

    rr = jnp.arange(4 * 2 * W)
    ph, i_loc, qj = rr // (2 * W), (rr % (2 * W)) // W, rr % W
    col = i_loc * lanes + (ph // 2) * W2 + 2 * qj + (ph % 2)
    pmat = jax.nn.one_hot(col, 2 * lanes, dtype=jnp.bfloat16)

    G = 2 if (N % 2 == 0 and T == 1) else 1
    grid = (N // G, T)
    cparams = pltpu.CompilerParams(
        dimension_semantics=("parallel", "parallel"),
        vmem_limit_bytes=48 * 1024 * 1024,
    )

    y_pre, stats = pl.pallas_call(
        functools.partial(_conv_stats_scatter_kernel,
                          rt=rt, W=W, Cin=Cin, Cout=Cout),
        out_shape=(jax.ShapeDtypeStruct((N, Cout, T * rt * lanes),
                                        jnp.bfloat16),
                   jax.ShapeDtypeStruct((N * T, 2, Cout), jnp.float32)),
        grid=grid,
        in_specs=[pl.BlockSpec((3, G, H + 2, W, Cin),
                               lambda n, t: (0, n, 0, 0, 0)),
                  pl.BlockSpec((16, Cin, Cout), lambda n, t: (0, 0, 0)),
                  pl.BlockSpec((4 * 2 * W, 2 * lanes), lambda n, t: (0, 0))],
        out_specs=(pl.BlockSpec((G, Cout, rt * lanes), lambda n, t: (n, 0, t)),
                   pl.BlockSpec((G, 2, Cout), lambda n, t: (n * T + t, 0, 0))),
        compiler_params=cparams,
    )(xp3, wf, pmat)

    m = float(N * H2 * W2)
    mean = jnp.sum(stats[:, 0, :], axis=0) / m
    var = jnp.maximum(jnp.sum(stats[:, 1, :], axis=0) / m - mean * mean, 0.0)
    scale = gamma.astype(jnp.float32) * jax.lax.rsqrt(var + eps)
    shift = beta.astype(jnp.float32) - mean * scale

    yb = jnp.maximum(y_pre.astype(jnp.float32) * scale[None, :, None]
                     + shift[None, :, None], 0.0).astype(jnp.bfloat16)
    y4 = yb.reshape(N, Cout, H2, W2).astype(jnp.float32)
    Hs, Ws = skip.shape[2], skip.shape[3]
    skip_p = jnp.concatenate(
        [jnp.zeros((N, Cskip, Hs, pad_size), skip.dtype), skip], axis=3)
    skip_p = jnp.concatenate(
        [skip_p, jnp.zeros((N, Cskip, pad_size, W2), skip.dtype)], axis=2)
    return jnp.concatenate([y4, skip_p], axis=1)


# speedup vs baseline: 1.0664x; 1.0019x over previous
import functools

import jax
import jax.numpy as jnp
from jax.experimental import pallas as pl
from jax.experimental.pallas import tpu as pltpu


def _conv_stats_scatter_kernel(xp_ref, w_ref, p_ref, y_ref, stats_ref,
                               *, rt, W, Cin, Cout):
    i0 = pl.program_id(1) * rt
    W2 = 2 * W
    G = y_ref.shape[0]
    for img in range(G):
        ssum = jnp.zeros((1, Cout), jnp.float32)
        qsum = jnp.zeros((1, Cout), jnp.float32)
        accs = []
        for a in range(2):
            for b in range(2):
                acc = jnp.zeros((rt * W, Cout), jnp.float32)
                for r in range(2):
                    for s in range(2):
                        xs = xp_ref[s + b, img, pl.ds(i0 + r + a, rt), :, :]
                        xf = xs.reshape(rt * W, Cin)
                        acc = acc + jnp.dot(
                            xf, w_ref[a * 8 + b * 4 + r * 2 + s],
                            preferred_element_type=jnp.float32)
                ssum = ssum + jnp.sum(acc, axis=0, keepdims=True)
                qsum = qsum + jnp.sum(acc * acc, axis=0, keepdims=True)
                accs.append(acc.astype(jnp.bfloat16))
        ys = []
        for g in range(rt // 2):
            lhs = jnp.concatenate(
                [ab[2 * W * g:2 * W * (g + 1)] for ab in accs], axis=0)
            ys.append(jax.lax.dot_general(lhs, p_ref[...],
                                          (((0,), (0,)), ((), ())),
                                          preferred_element_type=jnp.float32))
        stats_ref[img] = jnp.concatenate([ssum, qsum], axis=0)
        y_ref[img] = jnp.concatenate(ys, axis=1).astype(jnp.bfloat16)


def kernel(x, skip, conv_w, gamma, beta):
    N, Cin, H, W = x.shape
    Cout = conv_w.shape[0]
    Cskip = skip.shape[1]
    H2, W2 = 2 * H, 2 * W
    rt = 32 if H % 32 == 0 else 8
    T = H // rt
    lanes = 2 * W2
    eps = 0.8
    pad_size = 2

    xp = jnp.pad(jnp.transpose(x, (0, 2, 3, 1)),
                 ((0, 0), (1, 1), (1, 1), (0, 0)))
    xp3 = jnp.stack([xp[:, :, 0:W], xp[:, :, 1:W + 1], xp[:, :, 2:W + 2]],
                    axis=0).astype(jnp.bfloat16)
    sel = jnp.array([[[1.0, 0.0, 0.0], [0.0, 1.0, 1.0]],
                     [[1.0, 1.0, 0.0], [0.0, 0.0, 1.0]]], dtype=jnp.float32)
    wf = jnp.einsum("arh,bsw,oihw->abrsio", sel, sel,
                    conv_w.astype(jnp.float32))
    wf = wf.reshape(16, Cin, Cout).astype(jnp.bfloat16)

    rr = jnp.arange(4 * 2 * W)
    ph, i_loc, qj = rr // (2 * W), (rr % (2 * W)) // W, rr % W
    col = i_loc * lanes + (ph // 2) * W2 + 2 * qj + (ph % 2)
    pmat = jax.nn.one_hot(col, 2 * lanes, dtype=jnp.bfloat16)

    G = 4 if (N % 4 == 0 and T == 1) else 1
    grid = (N // G, T)
    cparams = pltpu.CompilerParams(
        dimension_semantics=("parallel", "parallel"),
        vmem_limit_bytes=48 * 1024 * 1024,
    )

    y_pre, stats = pl.pallas_call(
        functools.partial(_conv_stats_scatter_kernel,
                          rt=rt, W=W, Cin=Cin, Cout=Cout),
        out_shape=(jax.ShapeDtypeStruct((N, Cout, T * rt * lanes),
                                        jnp.bfloat16),
                   jax.ShapeDtypeStruct((N * T, 2, Cout), jnp.float32)),
        grid=grid,
        in_specs=[pl.BlockSpec((3, G, H + 2, W, Cin),
                               lambda n, t: (0, n, 0, 0, 0)),
                  pl.BlockSpec((16, Cin, Cout), lambda n, t: (0, 0, 0)),
                  pl.BlockSpec((4 * 2 * W, 2 * lanes), lambda n, t: (0, 0))],
        out_specs=(pl.BlockSpec((G, Cout, rt * lanes), lambda n, t: (n, 0, t)),
                   pl.BlockSpec((G, 2, Cout), lambda n, t: (n * T + t, 0, 0))),
        compiler_params=cparams,
    )(xp3, wf, pmat)

    m = float(N * H2 * W2)
    mean = jnp.sum(stats[:, 0, :], axis=0) / m
    var = jnp.maximum(jnp.sum(stats[:, 1, :], axis=0) / m - mean * mean, 0.0)
    scale = gamma.astype(jnp.float32) * jax.lax.rsqrt(var + eps)
    shift = beta.astype(jnp.float32) - mean * scale

    yb = jnp.maximum(y_pre.astype(jnp.float32) * scale[None, :, None]
                     + shift[None, :, None], 0.0).astype(jnp.bfloat16)
    y4 = yb.reshape(N, Cout, H2, W2).astype(jnp.float32)
    Hs, Ws = skip.shape[2], skip.shape[3]
    skip_p = jnp.concatenate(
        [jnp.zeros((N, Cskip, Hs, pad_size), skip.dtype), skip], axis=3)
    skip_p = jnp.concatenate(
        [skip_p, jnp.zeros((N, Cskip, pad_size, W2), skip.dtype)], axis=2)
    return jnp.concatenate([y4, skip_p], axis=1)
